# baseline (speedup 1.0000x reference)
import functools

import jax
import jax.numpy as jnp
from jax import lax
from jax.experimental import pallas as pl
from jax.experimental.pallas import tpu as pltpu
from jax.experimental.pallas import tpu_sc as plsc

N = 5000
NUM_CLASSES = 80
CONF_T = 0.5
NMS_T = 0.3
SIZE = 416.0
OFF = 4096.0

L = 16
NP = 5120
NPAD = 6400
NSEG = NPAD // L
NW = 32


def _decode_body(pred_ref, out_ref):
    p = pred_ref[...]
    cx = p[:, 0:1]
    cy = p[:, 1:2]
    w = p[:, 2:3]
    h = p[:, 3:4]
    obj = p[:, 4:5]
    cls = p[:, 5:]
    x1 = (cx - w * 0.5) * SIZE
    y1 = (cy - h * 0.5) * SIZE
    x2 = (cx + w * 0.5) * SIZE
    y2 = (cy + h * 0.5) * SIZE
    maxv = jnp.max(cls, axis=1, keepdims=True)
    ids = lax.broadcasted_iota(jnp.int32, cls.shape, 1)
    cid = jnp.min(jnp.where(cls == maxv, ids, NUM_CLASSES), axis=1, keepdims=True)
    offs = cid.astype(jnp.float32) * OFF
    conf = obj * maxv
    x1o = x1 + offs
    y1o = y1 + offs
    x2o = x2 + offs
    y2o = y2 + offs
    area = jnp.maximum(x2o - x1o, 0.0) * jnp.maximum(y2o - y1o, 0.0)
    out_ref[:, 0:1] = x1o
    out_ref[:, 1:2] = y1o
    out_ref[:, 2:3] = x2o
    out_ref[:, 3:4] = y2o
    out_ref[:, 4:5] = x1
    out_ref[:, 5:6] = y1
    out_ref[:, 6:7] = x2
    out_ref[:, 7:8] = y2
    out_ref[:, 8:9] = conf
    out_ref[:, 9:10] = area
    out_ref[:, 10:11] = cid.astype(jnp.float32)
    out_ref[:, 11:] = jnp.zeros_like(p[:, 11:16])


def _rank_body(ssp_ref, d_ref, starts_ref, counts_ref, run_ref):
    nb = NP // 128
    lane = lax.broadcasted_iota(jnp.int32, (128, 128), 1).astype(jnp.float32)
    sub = lax.broadcasted_iota(jnp.int32, (128, 128), 0).astype(jnp.float32)
    stril = (sub > lane).astype(jnp.float32)
    run_ref[...] = jnp.zeros((1, 128), jnp.float32)

    def blk(b, _):
        rows = ssp_ref[pl.ds(b * 128, 128), :]
        conf = rows[:, 8:9]
        cidf = rows[:, 10:11]
        keyf = jnp.where(conf > CONF_T, cidf, float(NUM_CLASSES))
        oh = jnp.where(lane == keyf, 1.0, 0.0)
        prev = jnp.dot(stril, oh, preferred_element_type=jnp.float32)
        rank = jnp.sum((prev + run_ref[...]) * oh, axis=1, keepdims=True)
        d_ref[pl.ds(b * 128, 128), :] = rank
        run_ref[...] = run_ref[...] + jnp.sum(oh, axis=0, keepdims=True)
        return 0

    lax.fori_loop(0, nb, blk, 0)
    counts = run_ref[...]
    cmask = lane[0:1, :] < float(NUM_CLASSES)
    seg = jnp.where(cmask, jnp.ceil(counts / L) * L, 0.0)
    striu = (sub < lane).astype(jnp.float32)
    starts = jnp.dot(seg, striu, preferred_element_type=jnp.float32)
    counts_ref[...] = jnp.where(cmask, counts, 0.0).astype(jnp.int32)
    starts_ref[...] = jnp.where(cmask, starts, 0.0).astype(jnp.int32)

    def blk2(b, _):
        rows = ssp_ref[pl.ds(b * 128, 128), :]
        conf = rows[:, 8:9]
        cidf = rows[:, 10:11]
        keyf = jnp.where(conf > CONF_T, cidf, float(NUM_CLASSES))
        oh = jnp.where(lane == keyf, 1.0, 0.0)
        st = jnp.sum(jnp.where(cmask, starts, 0.0) * oh, axis=1, keepdims=True)
        rank = d_ref[pl.ds(b * 128, 128), :]
        dv = jnp.where(keyf < float(NUM_CLASSES), st + rank, float(NPAD))
        d_ref[pl.ds(b * 128, 128), :] = dv
        return 0

    lax.fori_loop(0, nb, blk2, 0)


def _lane(v, i):
    idx = jnp.zeros((L,), jnp.int32) + i
    return v.at[idx].get(mode="promise_in_bounds")


def _iou_gt(x1a, y1a, x2a, y2a, aa, x1b, y1b, x2b, y2b, ab):
    xx1 = jnp.maximum(x1a, x1b)
    yy1 = jnp.maximum(y1a, y1b)
    xx2 = jnp.minimum(x2a, x2b)
    yy2 = jnp.minimum(y2a, y2b)
    inter = jnp.maximum(xx2 - xx1, 0.0) * jnp.maximum(yy2 - yy1, 0.0)
    union = aa + ab - inter
    iou = inter / jnp.maximum(union, 1e-9)
    return iou > NMS_T


def _sc_nms_body(gh, sth, cth, outh,
                 vx1, vy1, vx2, vy2, var, vst, vct, sup, gk):
    wid = lax.axis_index("s") * 2 + lax.axis_index("c")
    iota = lax.broadcasted_iota(jnp.int32, (L,), 0)
    zf = jnp.zeros((L,), jnp.float32)

    pltpu.sync_copy(gh.at[0], vx1)
    pltpu.sync_copy(gh.at[1], vy1)
    pltpu.sync_copy(gh.at[2], vx2)
    pltpu.sync_copy(gh.at[3], vy2)
    pltpu.sync_copy(gh.at[4], var)
    pltpu.sync_copy(sth, vst)
    pltpu.sync_copy(cth, vct)

    def scalar_at(ref, i):
        chunk = ref[pl.ds((i // L) * L, L)]
        v = jnp.where(iota == i % L, chunk, 0)
        for d in (8, 4, 2, 1):
            v = v + v.at[iota ^ d].get(mode="promise_in_bounds")
        return v[0]

    def run_class(c):
        start = scalar_at(vst, c)
        nc = scalar_at(vct, c)
        nch = (nc + L - 1) // L

        def zseg(j, _):
            sup[pl.ds(start + j * L, L)] = zf
            return 0

        lax.fori_loop(0, nch, zseg, 0)

        def chunk_body(ci, _):
            cb = start + ci * L
            bx1 = vx1[pl.ds(cb, L)]
            by1 = vy1[pl.ds(cb, L)]
            bx2 = vx2[pl.ds(cb, L)]
            by2 = vy2[pl.ds(cb, L)]
            bar = var[pl.ds(cb, L)]
            supc = sup[pl.ds(cb, L)]
            supc = jnp.where(ci * L + iota < nc, supc, 1.0)
            keepc = zf
            for l in range(L):
                kv = 1.0 - _lane(supc, l)
                ov = _iou_gt(_lane(bx1, l), _lane(by1, l), _lane(bx2, l),
                             _lane(by2, l), _lane(bar, l),
                             bx1, by1, bx2, by2, bar)
                ovf = jnp.where(ov, 1.0, 0.0)
                lf = jnp.where(iota > l, 1.0, 0.0)
                supc = jnp.maximum(supc, kv * lf * ovf)
                onef = jnp.where(iota == l, 1.0, 0.0)
                keepc = jnp.maximum(keepc, kv * onef)
            gk[pl.ds(cb, L)] = keepc

            def later_body(cj, _):
                jb = start + cj * L
                cx1 = vx1[pl.ds(jb, L)]
                cy1 = vy1[pl.ds(jb, L)]
                cx2 = vx2[pl.ds(jb, L)]
                cy2 = vy2[pl.ds(jb, L)]
                car = var[pl.ds(jb, L)]
                supj = sup[pl.ds(jb, L)]
                for l in range(L):
                    kl = _lane(keepc, l)
                    ov = _iou_gt(_lane(bx1, l), _lane(by1, l), _lane(bx2, l),
                                 _lane(by2, l), _lane(bar, l),
                                 cx1, cy1, cx2, cy2, car)
                    supj = jnp.maximum(supj, kl * jnp.where(ov, 1.0, 0.0))
                sup[pl.ds(jb, L)] = supj
                return 0

            lax.fori_loop(ci + 1, nch, later_body, 0)
            return 0

        lax.fori_loop(0, nch, chunk_body, 0)

        def pub(j, _):
            off = pl.multiple_of(start + j * L, L)
            pltpu.sync_copy(gk.at[pl.ds(off, L)], outh.at[pl.ds(off, L)])
            return 0

        lax.fori_loop(0, nch, pub, 0)

    run_class(wid)
    run_class(wid + NW)
    run_class(wid + 2 * NW)


def _sc_nms(grouped, starts, cnts):
    f = functools.partial(
        pl.kernel,
        mesh=plsc.VectorSubcoreMesh(core_axis_name="c", subcore_axis_name="s"),
        out_type=jax.ShapeDtypeStruct((NPAD,), jnp.float32),
        scratch_types=[pltpu.VMEM((NPAD,), jnp.float32)] * 5
        + [pltpu.VMEM((128,), jnp.int32)] * 2
        + [pltpu.VMEM((NPAD,), jnp.float32)] * 2,
    )(_sc_nms_body)
    return f(grouped, starts, cnts)


def kernel(predictions):
    pred = predictions[0]
    s1 = pl.pallas_call(
        _decode_body,
        out_shape=jax.ShapeDtypeStruct((N, 16), jnp.float32),
    )(pred)
    conf = s1[:, 8]
    order = jnp.argsort(-conf)
    order_p = jnp.concatenate([order, jnp.arange(N, NP, dtype=order.dtype)])
    s1p = jnp.pad(s1, ((0, NP - N), (0, 0)))
    ssp = s1p[order_p]

    df, starts128, counts128 = pl.pallas_call(
        _rank_body,
        out_shape=[
            jax.ShapeDtypeStruct((NP, 1), jnp.float32),
            jax.ShapeDtypeStruct((1, 128), jnp.int32),
            jax.ShapeDtypeStruct((1, 128), jnp.int32),
        ],
        scratch_shapes=[pltpu.VMEM((1, 128), jnp.float32)],
    )(ssp)
    d = df[:, 0].astype(jnp.int32)

    cols = jnp.concatenate(
        [ssp[:, 0:4], ssp[:, 9:10], jnp.zeros((NP, 3), jnp.float32)], axis=1
    )
    grouped = jnp.zeros((NPAD + 1, 8), jnp.float32).at[d].set(cols)
    gt = grouped[:NPAD].T

    keepg = _sc_nms(gt, starts128[0], counts128[0])
    kg = jnp.concatenate([keepg, jnp.zeros((1,), jnp.float32)])
    keepv = kg[d][:N]
    out = jnp.concatenate([ssp[:N, 4:8], ssp[:N, 8:9]], axis=1) * keepv[:, None]
    return out


# speedup vs baseline: 1.0340x; 1.0340x over previous
import functools

import jax
import jax.numpy as jnp
from jax import lax
from jax.experimental import pallas as pl
from jax.experimental.pallas import tpu as pltpu
from jax.experimental.pallas import tpu_sc as plsc

N = 5000
NUM_CLASSES = 80
CONF_T = 0.5
NMS_T = 0.3
SIZE = 416.0
OFF = 4096.0

L = 16
NP = 5120
NPAD = 6400
NSEG = NPAD // L
NW = 32


def _decode_body(pred_ref, out_ref):
    p = pred_ref[...]
    cx = p[:, 0:1]
    cy = p[:, 1:2]
    w = p[:, 2:3]
    h = p[:, 3:4]
    obj = p[:, 4:5]
    cls = p[:, 5:]
    x1 = (cx - w * 0.5) * SIZE
    y1 = (cy - h * 0.5) * SIZE
    x2 = (cx + w * 0.5) * SIZE
    y2 = (cy + h * 0.5) * SIZE
    maxv = jnp.max(cls, axis=1, keepdims=True)
    ids = lax.broadcasted_iota(jnp.int32, cls.shape, 1)
    cid = jnp.min(jnp.where(cls == maxv, ids, NUM_CLASSES), axis=1, keepdims=True)
    offs = cid.astype(jnp.float32) * OFF
    conf = obj * maxv
    x1o = x1 + offs
    y1o = y1 + offs
    x2o = x2 + offs
    y2o = y2 + offs
    area = jnp.maximum(x2o - x1o, 0.0) * jnp.maximum(y2o - y1o, 0.0)
    out_ref[:, 0:1] = x1o
    out_ref[:, 1:2] = y1o
    out_ref[:, 2:3] = x2o
    out_ref[:, 3:4] = y2o
    out_ref[:, 4:5] = x1
    out_ref[:, 5:6] = y1
    out_ref[:, 6:7] = x2
    out_ref[:, 7:8] = y2
    out_ref[:, 8:9] = conf
    out_ref[:, 9:10] = area
    out_ref[:, 10:11] = cid.astype(jnp.float32)
    out_ref[:, 11:] = jnp.zeros_like(p[:, 11:16])


def _rank_body(ssp_ref, d_ref, starts_ref, counts_ref, run_ref):
    nb = NP // 128
    lane = lax.broadcasted_iota(jnp.int32, (128, 128), 1).astype(jnp.float32)
    sub = lax.broadcasted_iota(jnp.int32, (128, 128), 0).astype(jnp.float32)
    stril = (sub > lane).astype(jnp.float32)
    run_ref[...] = jnp.zeros((1, 128), jnp.float32)

    def blk(b, _):
        rows = ssp_ref[pl.ds(b * 128, 128), :]
        conf = rows[:, 8:9]
        cidf = rows[:, 10:11]
        keyf = jnp.where(conf > CONF_T, cidf, float(NUM_CLASSES))
        oh = jnp.where(lane == keyf, 1.0, 0.0)
        prev = jnp.dot(stril, oh, preferred_element_type=jnp.float32)
        rank = jnp.sum((prev + run_ref[...]) * oh, axis=1, keepdims=True)
        d_ref[pl.ds(b * 128, 128), :] = rank
        run_ref[...] = run_ref[...] + jnp.sum(oh, axis=0, keepdims=True)
        return 0

    lax.fori_loop(0, nb, blk, 0)
    counts = run_ref[...]
    cmask = lane[0:1, :] < float(NUM_CLASSES)
    seg = jnp.where(cmask, jnp.ceil(counts / L) * L, 0.0)
    striu = (sub < lane).astype(jnp.float32)
    starts = jnp.dot(seg, striu, preferred_element_type=jnp.float32)
    counts_ref[...] = jnp.where(cmask, counts, 0.0).astype(jnp.int32)
    starts_ref[...] = jnp.where(cmask, starts, 0.0).astype(jnp.int32)

    def blk2(b, _):
        rows = ssp_ref[pl.ds(b * 128, 128), :]
        conf = rows[:, 8:9]
        cidf = rows[:, 10:11]
        keyf = jnp.where(conf > CONF_T, cidf, float(NUM_CLASSES))
        oh = jnp.where(lane == keyf, 1.0, 0.0)
        st = jnp.sum(jnp.where(cmask, starts, 0.0) * oh, axis=1, keepdims=True)
        rank = d_ref[pl.ds(b * 128, 128), :]
        dv = jnp.where(keyf < float(NUM_CLASSES), st + rank, float(NPAD))
        d_ref[pl.ds(b * 128, 128), :] = dv
        return 0

    lax.fori_loop(0, nb, blk2, 0)


def _lane(v, i):
    idx = jnp.zeros((L,), jnp.int32) + i
    return v.at[idx].get(mode="promise_in_bounds")


def _iou_gt(x1a, y1a, x2a, y2a, aa, x1b, y1b, x2b, y2b, ab):
    xx1 = jnp.maximum(x1a, x1b)
    yy1 = jnp.maximum(y1a, y1b)
    xx2 = jnp.minimum(x2a, x2b)
    yy2 = jnp.minimum(y2a, y2b)
    inter = jnp.maximum(xx2 - xx1, 0.0) * jnp.maximum(yy2 - yy1, 0.0)
    union = aa + ab - inter
    iou = inter / jnp.maximum(union, 1e-9)
    return iou > NMS_T


def _sc_nms_body(gh, sth, cth, outh,
                 vx1, vy1, vx2, vy2, var, vst, vct, sup, gk, sem):
    wid = lax.axis_index("s") * 2 + lax.axis_index("c")
    iota = lax.broadcasted_iota(jnp.int32, (L,), 0)
    zf = jnp.zeros((L,), jnp.float32)

    cps = [pltpu.async_copy(gh.at[0], vx1, sem),
           pltpu.async_copy(gh.at[1], vy1, sem),
           pltpu.async_copy(gh.at[2], vx2, sem),
           pltpu.async_copy(gh.at[3], vy2, sem),
           pltpu.async_copy(gh.at[4], var, sem),
           pltpu.async_copy(sth, vst, sem),
           pltpu.async_copy(cth, vct, sem)]
    for cp in cps:
        cp.wait()

    def scalar_at(ref, i):
        chunk = ref[pl.ds((i // L) * L, L)]
        v = jnp.where(iota == i % L, chunk, 0)
        for d in (8, 4, 2, 1):
            v = v + v.at[iota ^ d].get(mode="promise_in_bounds")
        return v[0]

    def run_class(c):
        start = scalar_at(vst, c)
        nc = scalar_at(vct, c)
        nch = (nc + L - 1) // L

        def zseg(j, _):
            sup[pl.ds(start + j * L, L)] = zf
            return 0

        lax.fori_loop(0, nch, zseg, 0)

        def chunk_body(ci, _):
            cb = start + ci * L
            bx1 = vx1[pl.ds(cb, L)]
            by1 = vy1[pl.ds(cb, L)]
            bx2 = vx2[pl.ds(cb, L)]
            by2 = vy2[pl.ds(cb, L)]
            bar = var[pl.ds(cb, L)]
            supc = sup[pl.ds(cb, L)]
            supc = jnp.where(ci * L + iota < nc, supc, 1.0)
            keepc = zf
            for l in range(L):
                kv = 1.0 - _lane(supc, l)
                ov = _iou_gt(_lane(bx1, l), _lane(by1, l), _lane(bx2, l),
                             _lane(by2, l), _lane(bar, l),
                             bx1, by1, bx2, by2, bar)
                ovf = jnp.where(ov, 1.0, 0.0)
                lf = jnp.where(iota > l, 1.0, 0.0)
                supc = jnp.maximum(supc, kv * lf * ovf)
                onef = jnp.where(iota == l, 1.0, 0.0)
                keepc = jnp.maximum(keepc, kv * onef)
            gk[pl.ds(cb, L)] = keepc

            def later_body(cj, _):
                jb = start + cj * L
                cx1 = vx1[pl.ds(jb, L)]
                cy1 = vy1[pl.ds(jb, L)]
                cx2 = vx2[pl.ds(jb, L)]
                cy2 = vy2[pl.ds(jb, L)]
                car = var[pl.ds(jb, L)]
                supj = sup[pl.ds(jb, L)]
                for l in range(L):
                    kl = _lane(keepc, l)
                    ov = _iou_gt(_lane(bx1, l), _lane(by1, l), _lane(bx2, l),
                                 _lane(by2, l), _lane(bar, l),
                                 cx1, cy1, cx2, cy2, car)
                    supj = jnp.maximum(supj, kl * jnp.where(ov, 1.0, 0.0))
                sup[pl.ds(jb, L)] = supj
                return 0

            lax.fori_loop(ci + 1, nch, later_body, 0)
            return 0

        lax.fori_loop(0, nch, chunk_body, 0)

        def pub(j, _):
            off = pl.multiple_of(start + j * L, L)
            pltpu.sync_copy(gk.at[pl.ds(off, L)], outh.at[pl.ds(off, L)])
            return 0

        lax.fori_loop(0, nch, pub, 0)

    run_class(wid)
    run_class(wid + NW)
    run_class(wid + 2 * NW)


def _sc_nms(grouped, starts, cnts):
    f = functools.partial(
        pl.kernel,
        mesh=plsc.VectorSubcoreMesh(core_axis_name="c", subcore_axis_name="s"),
        out_type=jax.ShapeDtypeStruct((NPAD,), jnp.float32),
        scratch_types=[pltpu.VMEM((NPAD,), jnp.float32)] * 5
        + [pltpu.VMEM((128,), jnp.int32)] * 2
        + [pltpu.VMEM((NPAD,), jnp.float32)] * 2
        + [pltpu.SemaphoreType.DMA],
    )(_sc_nms_body)
    return f(grouped, starts, cnts)


def kernel(predictions):
    pred = predictions[0]
    s1 = pl.pallas_call(
        _decode_body,
        out_shape=jax.ShapeDtypeStruct((N, 16), jnp.float32),
    )(pred)
    conf = s1[:, 8]
    order = jnp.argsort(-conf)
    order_p = jnp.concatenate([order, jnp.arange(N, NP, dtype=order.dtype)])
    s1p = jnp.pad(s1, ((0, NP - N), (0, 0)))
    ssp = s1p[order_p]

    df, starts128, counts128 = pl.pallas_call(
        _rank_body,
        out_shape=[
            jax.ShapeDtypeStruct((NP, 1), jnp.float32),
            jax.ShapeDtypeStruct((1, 128), jnp.int32),
            jax.ShapeDtypeStruct((1, 128), jnp.int32),
        ],
        scratch_shapes=[pltpu.VMEM((1, 128), jnp.float32)],
    )(ssp)
    d = df[:, 0].astype(jnp.int32)

    cols = jnp.concatenate(
        [ssp[:, 0:4], ssp[:, 9:10], jnp.zeros((NP, 3), jnp.float32)], axis=1
    )
    grouped = jnp.zeros((NPAD + 1, 8), jnp.float32).at[d].set(cols)
    gt = grouped[:NPAD].T

    keepg = _sc_nms(gt, starts128[0], counts128[0])
    kg = jnp.concatenate([keepg, jnp.zeros((1,), jnp.float32)])
    keepv = kg[d][:N]
    out = jnp.concatenate([ssp[:N, 4:8], ssp[:N, 8:9]], axis=1) * keepv[:, None]
    return out
